# Initial kernel scaffold; baseline (speedup 1.0000x reference)
import jax
import jax.numpy as jnp
from jax.experimental import pallas as pl


def kernel(pred_locs, pred_scores, priors):
    raise NotImplementedError("write your pallas kernel here")


# speedup vs baseline: 77.8728x; 77.8728x over previous
import jax
import jax.numpy as jnp
from jax.experimental import pallas as pl

N = 5000
NPAD = 5120
NB = 4
NC1 = 21
NCLS = 20
K = 200
MIN_SCORE = 0.01
OVERLAP = 0.45
INT_MIN = -(2 ** 31)
BIG = 2 ** 30


def _detect_kernel(logits_ref, locs_ref, priors_ref, out_ref):
    z = logits_ref[0]
    lx = locs_ref[0]
    pr = priors_ref[...]

    zmax = jnp.max(z, axis=0, keepdims=True)
    e = jnp.exp(z - zmax)
    ssum = jnp.sum(e, axis=0, keepdims=True)
    p = e / ssum
    pc = p[1:NC1]

    cx = pr[0:1]; cy = pr[1:2]; w = pr[2:3]; h = pr[3:4]
    tx = lx[0:1]; ty = lx[1:2]; tw = lx[2:3]; th = lx[3:4]
    dcx = cx + tx * 0.1 * w
    dcy = cy + ty * 0.1 * h
    dw = w * jnp.exp(tw * 0.2)
    dh = h * jnp.exp(th * 0.2)
    x1 = dcx - dw / 2.0
    y1 = dcy - dh / 2.0
    x2 = x1 + dw
    y2 = y1 + dh

    lane = jax.lax.broadcasted_iota(jnp.int32, (1, NPAD), 1)
    valid = (pc > MIN_SCORE) & (lane < N)
    key0 = jnp.where(valid, jax.lax.bitcast_convert_type(pc, jnp.int32),
                     INT_MIN)

    lane200 = jax.lax.broadcasted_iota(jnp.int32, (1, K), 1)

    def sel_body(t, st):
        key, ck, bx1, by1, bx2, by2 = st
        m = jnp.max(key, axis=1, keepdims=True)
        idx = jnp.max(jnp.where(key == m, lane, -1), axis=1, keepdims=True)
        oh = lane == idx
        key = jnp.where(oh, INT_MIN, key)
        tm = lane200 == t
        ck = jnp.where(tm, m, ck)
        gx1 = jnp.sum(jnp.where(oh, x1, 0.0), axis=1, keepdims=True)
        gy1 = jnp.sum(jnp.where(oh, y1, 0.0), axis=1, keepdims=True)
        gx2 = jnp.sum(jnp.where(oh, x2, 0.0), axis=1, keepdims=True)
        gy2 = jnp.sum(jnp.where(oh, y2, 0.0), axis=1, keepdims=True)
        bx1 = jnp.where(tm, gx1, bx1)
        by1 = jnp.where(tm, gy1, by1)
        bx2 = jnp.where(tm, gx2, bx2)
        by2 = jnp.where(tm, gy2, by2)
        return key, ck, bx1, by1, bx2, by2

    zk = jnp.zeros((NCLS, K), jnp.float32)
    st0 = (key0, jnp.full((NCLS, K), INT_MIN, jnp.int32), zk, zk, zk, zk)
    _, ck, bx1, by1, bx2, by2 = jax.lax.fori_loop(0, K, sel_body, st0)

    cvalid = ck != INT_MIN
    cs = jax.lax.bitcast_convert_type(ck, jnp.float32)
    area = (bx2 - bx1) * (by2 - by1)

    def nms_body(t, sup):
        tm = lane200 == t

        def pickf(a):
            return jnp.sum(jnp.where(tm, a, 0.0), axis=1, keepdims=True)

        def pickb(a):
            return jnp.sum(jnp.where(tm & a, 1, 0), axis=1, keepdims=True) > 0

        x1t = pickf(bx1); y1t = pickf(by1)
        x2t = pickf(bx2); y2t = pickf(by2)
        art = pickf(area)
        supt = jnp.sum(jnp.where(tm, sup, 0), axis=1, keepdims=True) > 0
        vt = pickb(cvalid)
        active = vt & jnp.logical_not(supt)
        xx1 = jnp.maximum(bx1, x1t)
        yy1 = jnp.maximum(by1, y1t)
        xx2 = jnp.minimum(jnp.maximum(bx2, 0.0), x2t)
        yy2 = jnp.minimum(jnp.maximum(by2, 0.0), y2t)
        ww = jnp.maximum(xx2 - xx1, 0.0)
        hh = jnp.maximum(yy2 - yy1, 0.0)
        inter = ww * hh
        union = area - inter + art
        iou = inter / union
        supnew = (lane200 > t) & (iou > OVERLAP)
        return jnp.where(active & supnew, 1, sup)

    sup = jax.lax.fori_loop(0, K, nms_body, jnp.zeros((NCLS, K), jnp.int32))
    kept = cvalid & (sup == 0)

    rowkey = jnp.where(kept, ck, INT_MIN)
    p0 = pc[:, 0:1]
    fx1 = x1[0:1, 0:1]; fy1 = y1[0:1, 0:1]
    fx2 = x2[0:1, 0:1]; fy2 = y2[0:1, 0:1]
    c_x1 = jnp.where(kept, bx1, fx1)
    c_y1 = jnp.where(kept, by1, fy1)
    c_x2 = jnp.where(kept, bx2, fx2)
    c_y2 = jnp.where(kept, by2, fy2)
    c_s = jnp.where(kept, cs, p0)
    sub20 = jax.lax.broadcasted_iota(jnp.int32, (NCLS, 1), 0)
    c_lab = jnp.zeros((NCLS, K), jnp.float32) + (sub20 + 1).astype(jnp.float32)

    def out_body(t, key):
        mlane = jnp.max(key, axis=1, keepdims=True)
        m = jnp.max(mlane, axis=0, keepdims=True)
        cbest = jnp.min(jnp.where(mlane == m, sub20, BIG), axis=0,
                        keepdims=True)
        rowmask = sub20 == cbest
        tb = jnp.where(rowmask & (key == m), lane200 + jnp.zeros((NCLS, K), jnp.int32), BIG)
        tbest = jnp.min(jnp.min(tb, axis=1, keepdims=True), axis=0,
                        keepdims=True)
        oh = rowmask & (lane200 == tbest)

        def ext(a):
            s = jnp.sum(jnp.where(oh, a, 0.0), axis=1, keepdims=True)
            return jnp.sum(s, axis=0, keepdims=True)

        out_ref[0, pl.ds(t, 1), 0:1] = ext(c_x1)
        out_ref[0, pl.ds(t, 1), 1:2] = ext(c_y1)
        out_ref[0, pl.ds(t, 1), 2:3] = ext(c_x2)
        out_ref[0, pl.ds(t, 1), 3:4] = ext(c_y2)
        out_ref[0, pl.ds(t, 1), 4:5] = ext(c_s)
        out_ref[0, pl.ds(t, 1), 5:6] = ext(c_lab)
        out_ref[0, pl.ds(t, 1), 6:8] = jnp.zeros((1, 2), jnp.float32)
        return jnp.where(oh, INT_MIN, key)

    jax.lax.fori_loop(0, K, out_body, rowkey)


def kernel(pred_locs, pred_scores, priors):
    logits_t = jnp.pad(jnp.transpose(pred_scores, (0, 2, 1)),
                       ((0, 0), (0, 0), (0, NPAD - N)))
    locs_t = jnp.pad(jnp.transpose(pred_locs, (0, 2, 1)),
                     ((0, 0), (0, 0), (0, NPAD - N)))
    priors_t = jnp.pad(priors.T, ((0, 0), (0, NPAD - N)))

    out = pl.pallas_call(
        _detect_kernel,
        grid=(NB,),
        in_specs=[
            pl.BlockSpec((1, NC1, NPAD), lambda b: (b, 0, 0)),
            pl.BlockSpec((1, 4, NPAD), lambda b: (b, 0, 0)),
            pl.BlockSpec((4, NPAD), lambda b: (0, 0)),
        ],
        out_specs=pl.BlockSpec((1, K, 8), lambda b: (b, 0, 0)),
        out_shape=jax.ShapeDtypeStruct((NB, K, 8), jnp.float32),
    )(logits_t, locs_t, priors_t)

    res = out[:, :, :6]
    return tuple(res[b] for b in range(NB))
